# Initial kernel scaffold; baseline (speedup 1.0000x reference)
import jax
import jax.numpy as jnp
from jax.experimental import pallas as pl


def kernel(x, lower_index, lower_values, upper_index, upper_values, W_irr, W_sol):
    raise NotImplementedError("write your pallas kernel here")


# speedup vs baseline: 3.5993x; 3.5993x over previous
import functools

import jax
import jax.numpy as jnp
from jax import lax
from jax.experimental import pallas as pl
from jax.experimental.pallas import tpu as pltpu
from jax.experimental.pallas import tpu_sc as plsc

N = 10000
E = 320000
D = 128

NC = 2
NS = 16
L = 16

CH = 80
EPT = E // NS
NCH = EPT // CH
GRP = 8
NCHP = 256
RPT = 624
REM = N - NS * RPT


def _matmul_body(x_ref, w_ref, y_ref):
    y_ref[0] = jnp.dot(x_ref[...], w_ref[0], preferred_element_type=jnp.float32)


def _projections(x, W2):
    grid = (2, 25)
    br = N // 25
    return pl.pallas_call(
        _matmul_body,
        grid=grid,
        in_specs=[
            pl.BlockSpec((br, D), lambda g, i: (i, 0)),
            pl.BlockSpec((1, D, D), lambda g, i: (g, 0, 0)),
        ],
        out_specs=pl.BlockSpec((1, br, D), lambda g, i: (g, i, 0)),
        out_shape=jax.ShapeDtypeStruct((2, N, D), jnp.float32),
    )(x, W2)


_BCAST_DN = lax.GatherDimensionNumbers(
    offset_dims=(), collapsed_slice_dims=(0,), start_index_map=(0,))


def _sc_body(y_hbm, src_hbm, dst_hbm, val_hbm, part_hbm,
             sidx_v, didx_v, val_v, rows_v, acc_sh, sem):
    c = lax.axis_index("c")
    s = lax.axis_index("s")

    def zrow(r, carry):
        for k in range(D // L):
            rows_v[r, pl.ds(k * L, L)] = jnp.zeros((L,), jnp.float32)
        return carry
    lax.fori_loop(0, CH, zrow, 0)
    for i in range(RPT // CH):
        pltpu.sync_copy(rows_v, acc_sh.at[pl.ds(s * RPT + i * CH, CH)])
    zr = RPT - (RPT // CH) * CH
    pltpu.sync_copy(rows_v.at[pl.ds(0, zr)],
                    acc_sh.at[pl.ds(s * RPT + RPT - zr, zr)])

    @pl.when(s == 0)
    def _zero_rem():
        pltpu.sync_copy(rows_v.at[pl.ds(0, REM)], acc_sh.at[pl.ds(NS * RPT, REM)])

    plsc.subcore_barrier()

    off = c * N
    zlane = lax.iota(jnp.int32, L) * 0

    def grp_body(gg, carry):
        pltpu.sync_copy(src_hbm.at[c, s, pl.ds(gg * GRP, GRP)], sidx_v)
        pltpu.sync_copy(dst_hbm.at[c, s, pl.ds(gg * GRP, GRP)], didx_v)
        pltpu.sync_copy(val_hbm.at[c, s, pl.ds(gg * GRP, GRP)], val_v)

        def chunk(j, carry2):
            for k in range(CH // L):
                sidx_v[j, pl.ds(k * L, L)] = sidx_v[j, pl.ds(k * L, L)] + off
            pltpu.async_copy(y_hbm.at[sidx_v.at[j]], rows_v, sem).wait()

            def group(g, carry3):
                vals16 = val_v[j, pl.ds(g * L, L)]
                for e in range(L):
                    vb = lax.gather(
                        vals16, (zlane + e).reshape(L, 1), _BCAST_DN,
                        slice_sizes=(1,),
                        mode=lax.GatherScatterMode.PROMISE_IN_BOUNDS)
                    row = g * L + e
                    for k in range(D // L):
                        rows_v[row, pl.ds(k * L, L)] = (
                            rows_v[row, pl.ds(k * L, L)] * vb)
                return carry3
            lax.fori_loop(0, CH // L, group, 0)

            pltpu.sync_copy(rows_v, acc_sh.at[didx_v.at[j]], add=True)
            return carry2
        lax.fori_loop(0, GRP, chunk, 0)
        return carry
    lax.fori_loop(0, NCHP // GRP, grp_body, 0)

    plsc.subcore_barrier()
    pltpu.sync_copy(acc_sh.at[pl.ds(s * RPT, RPT)],
                    part_hbm.at[c, pl.ds(s * RPT, RPT)])

    @pl.when(s == 0)
    def _copy_rem():
        pltpu.sync_copy(acc_sh.at[pl.ds(NS * RPT, REM)],
                        part_hbm.at[c, pl.ds(NS * RPT, REM)])


def _sparse_partials(y2, src2, dst2, val2):
    mesh = plsc.VectorSubcoreMesh(
        core_axis_name="c", subcore_axis_name="s", num_cores=NC, num_subcores=NS)
    fn = pl.kernel(
        _sc_body,
        out_type=jax.ShapeDtypeStruct((NC, N, D), jnp.float32),
        mesh=mesh,
        scratch_types=[
            pltpu.VMEM((GRP, CH), jnp.int32),
            pltpu.VMEM((GRP, CH), jnp.int32),
            pltpu.VMEM((GRP, CH), jnp.float32),
            pltpu.VMEM((CH, D), jnp.float32),
            pltpu.VMEM_SHARED((N, D), jnp.float32),
            pltpu.SemaphoreType.DMA,
        ],
    )
    return fn(y2.reshape(2 * N, D), src2, dst2, val2)


def _add_body(p_ref, o_ref):
    o_ref[...] = p_ref[0] + p_ref[1]


def _sum_partials(part):
    br = N // 25
    return pl.pallas_call(
        _add_body,
        grid=(25,),
        in_specs=[pl.BlockSpec((2, br, D), lambda i: (0, i, 0))],
        out_specs=pl.BlockSpec((br, D), lambda i: (i, 0)),
        out_shape=jax.ShapeDtypeStruct((N, D), jnp.float32),
    )(part)


def kernel(x, lower_index, lower_values, upper_index, upper_values, W_irr, W_sol):
    W2 = jnp.stack([W_irr, W_sol])
    y2 = _projections(x, W2)
    pad = ((0, 0), (0, 0), (0, NCHP - NCH), (0, 0))
    src2 = jnp.pad(
        jnp.stack([lower_index[1], upper_index[1]]).reshape(2, NS, NCH, CH), pad)
    dst2 = jnp.pad(
        jnp.stack([lower_index[0], upper_index[0]]).reshape(2, NS, NCH, CH), pad)
    val2 = jnp.pad(
        jnp.stack([lower_values, upper_values]).reshape(2, NS, NCH, CH), pad)
    part = _sparse_partials(y2, src2, dst2, val2)
    return _sum_partials(part)
